# baseline (speedup 1.0000x reference)
import jax
import jax.numpy as jnp
from jax.experimental import pallas as pl
from jax.experimental.pallas import tpu as pltpu

_BT = 256


def _tc_body(len_ref, in_ref, out_ref):
    b = pl.program_id(0)
    j = pl.program_id(1)
    nl = (len_ref[b] + 1) // 2
    row = jax.lax.broadcasted_iota(jnp.int32, (_BT, 1), 0) + j * _BT
    mask = (row < nl).astype(jnp.float32)
    out_ref[0, :, :] = in_ref[0, :, 0:1024] * mask


def kernel(values, lengths):
    B, T, D = values.shape
    T2 = T // 2
    lengths = lengths.astype(jnp.int32)
    v3 = values.reshape(B, T2, 2 * D)

    def in_map(b, j, len_ref):
        nl = (len_ref[b] + 1) // 2
        jmax = jnp.maximum(pl.cdiv(nl, _BT) - 1, 0)
        return (b, jnp.minimum(j, jmax), 0)

    def out_map(b, j, len_ref):
        return (b, j, 0)

    out = pl.pallas_call(
        _tc_body,
        grid_spec=pltpu.PrefetchScalarGridSpec(
            num_scalar_prefetch=1,
            grid=(B, T2 // _BT),
            in_specs=[pl.BlockSpec((1, _BT, 2 * D), in_map)],
            out_specs=pl.BlockSpec((1, _BT, D), out_map),
        ),
        out_shape=jax.ShapeDtypeStruct((B, T2, D), jnp.float32),
    )(lengths, v3)
    new_lengths = (lengths + 1) // 2
    return out, new_lengths


# speedup vs baseline: 1.1117x; 1.1117x over previous
import jax
import jax.numpy as jnp
from jax.experimental import pallas as pl
from jax.experimental.pallas import tpu as pltpu

_BT = 1024


def _tc_body(len_ref, in_ref, out_ref):
    b = pl.program_id(0)
    j = pl.program_id(1)
    nl = (len_ref[b] + 1) // 2
    row = jax.lax.broadcasted_iota(jnp.int32, (_BT, 1), 0) + j * _BT
    mask = (row < nl).astype(jnp.float32)
    out_ref[0, :, :] = in_ref[0, :, :] * mask


def kernel(values, lengths):
    B, T, D = values.shape
    T2 = T // 2
    lengths = lengths.astype(jnp.int32)
    v3 = values.reshape(B, T2, 2 * D)

    def in_map(b, j, len_ref):
        nl = (len_ref[b] + 1) // 2
        jmax = jnp.maximum(pl.cdiv(nl, _BT) - 1, 0)
        return (b, jnp.minimum(j, jmax), 0)

    def out_map(b, j, len_ref):
        return (b, j, 0)

    out = pl.pallas_call(
        _tc_body,
        grid_spec=pltpu.PrefetchScalarGridSpec(
            num_scalar_prefetch=1,
            grid=(B, T2 // _BT),
            in_specs=[pl.BlockSpec((1, _BT, D), in_map)],
            out_specs=pl.BlockSpec((1, _BT, D), out_map),
        ),
        out_shape=jax.ShapeDtypeStruct((B, T2, D), jnp.float32),
    )(lengths, v3)
    new_lengths = (lengths + 1) // 2
    return out, new_lengths
